# baseline (speedup 1.0000x reference)
import functools

import jax
import jax.numpy as jnp
from jax import lax
from jax.experimental import pallas as pl
from jax.experimental.pallas import tpu as pltpu
from jax.experimental.pallas import tpu_sc as plsc

V = 1000
E = 64
VP = 1024
NC = 2
NS = 16
NW = NC * NS
BW = 32


def _fused_table(emb, w_pad, b_pad):
    def mm(e_ref, w_ref, b_ref, o_ref):
        o_ref[...] = lax.dot_general(
            e_ref[...], w_ref[...],
            dimension_numbers=(((1,), (1,)), ((), ())),
            preferred_element_type=jnp.float32,
        ) + b_ref[...]

    return pl.pallas_call(
        mm,
        out_shape=jax.ShapeDtypeStruct((V, VP), jnp.float32),
    )(emb, w_pad, b_pad)


def _make_gather(bsz, seq):
    mesh = plsc.VectorSubcoreMesh(core_axis_name="c", subcore_axis_name="s")

    @functools.partial(
        pl.kernel, mesh=mesh,
        out_type=jax.ShapeDtypeStruct((seq, bsz, VP), jnp.float32),
        scratch_types=[
            pltpu.VMEM((seq, BW), jnp.int32),
            pltpu.VMEM((BW, VP), jnp.float32),
            pltpu.VMEM((BW, VP), jnp.float32),
            pltpu.SemaphoreType.DMA,
            pltpu.SemaphoreType.DMA,
            pltpu.SemaphoreType.DMA,
            pltpu.SemaphoreType.DMA,
        ],
    )
    def gather_k(p_hbm, idx_hbm, out_hbm, idx_v, rows_a, rows_b,
                 sem_ga, sem_gb, sem_wa, sem_wb):
        wid = lax.axis_index("s") * NC + lax.axis_index("c")
        base = wid * BW
        pltpu.sync_copy(idx_hbm.at[wid], idx_v)

        def gather_descr(l, rows, sem):
            return pltpu.make_async_copy(
                p_hbm.at[idx_v.at[l]], rows, sem)

        def write_descr(l, rows, sem):
            return pltpu.make_async_copy(
                rows, out_hbm.at[l, pl.ds(base, BW), :], sem)

        def issue_gather(l, rows, sem):
            pltpu.async_copy(p_hbm.at[idx_v.at[l]], rows, sem)

        issue_gather(0, rows_a, sem_ga)
        issue_gather(1, rows_b, sem_gb)

        def step(l, rows, sem_g, sem_w, nxt_ok):
            gather_descr(l, rows, sem_g).wait()
            pltpu.async_copy(rows, out_hbm.at[l, pl.ds(base, BW), :], sem_w)
            write_descr(l, rows, sem_w).wait()

            @pl.when(nxt_ok)
            def _():
                issue_gather(l + 2, rows, sem_g)

        def body(j, carry):
            l0 = 2 * j
            ok = j + 1 < seq // 2
            step(l0, rows_a, sem_ga, sem_wa, ok)
            step(l0 + 1, rows_b, sem_gb, sem_wb, ok)
            return carry

        lax.fori_loop(0, seq // 2, body, 0)

    return gather_k


def _transpose_q(inter, eye, seq, bsz):
    def tr(i_ref, e_ref, o_ref):
        y = lax.dot_general(
            i_ref[0], e_ref[...],
            dimension_numbers=(((0,), (0,)), ((), ())),
            preferred_element_type=jnp.float32,
        )
        o_ref[0] = y[:V, :]

    return pl.pallas_call(
        tr,
        grid=(seq, bsz // 128),
        in_specs=[
            pl.BlockSpec((1, 128, VP), lambda l, bb: (l, bb, 0)),
            pl.BlockSpec((128, 128), lambda l, bb: (0, 0)),
        ],
        out_specs=pl.BlockSpec((1, V, 128), lambda l, bb: (l, 0, bb)),
        out_shape=jax.ShapeDtypeStruct((seq, V, bsz), jnp.float32),
    )(inter, eye)


def kernel(input_ids, embedding_table, linear_w, linear_b):
    bsz, seq = input_ids.shape
    w_pad = jnp.pad(linear_w, ((0, VP - V), (0, 0)))
    b_pad = jnp.pad(linear_b, (0, VP - V)).reshape(1, VP)
    p = _fused_table(embedding_table, w_pad, b_pad)
    ids3 = (input_ids.astype(jnp.int32)
            .reshape(NW, BW, seq).transpose(0, 2, 1))
    inter = _make_gather(bsz, seq)(p, ids3)
    eye = jnp.eye(128, dtype=jnp.float32)
    q = _transpose_q(inter, eye, seq, bsz)
    return lax.transpose(q, (2, 0, 1))


# speedup vs baseline: 1.5543x; 1.5543x over previous
import functools

import jax
import jax.numpy as jnp
from jax import lax
from jax.experimental import pallas as pl
from jax.experimental.pallas import tpu as pltpu
from jax.experimental.pallas import tpu_sc as plsc

V = 1000
E = 64
VP = 1024
NC = 2
NS = 16
NW = NC * NS
BW = 32
LCH = 10
BBLK = 512


def _fused_table(emb, w_pad, b_pad):
    def mm(e_ref, w_ref, b_ref, o_ref):
        o_ref[...] = lax.dot_general(
            e_ref[...], w_ref[...],
            dimension_numbers=(((1,), (1,)), ((), ())),
            preferred_element_type=jnp.float32,
        ) + b_ref[...]

    return pl.pallas_call(
        mm,
        out_shape=jax.ShapeDtypeStruct((V, VP), jnp.float32),
    )(emb, w_pad, b_pad)


def _make_gather(bsz, nl):
    mesh = plsc.VectorSubcoreMesh(core_axis_name="c", subcore_axis_name="s")

    @functools.partial(
        pl.kernel, mesh=mesh,
        out_type=jax.ShapeDtypeStruct((nl, bsz, VP), jnp.float32),
        scratch_types=[
            pltpu.VMEM((nl, BW), jnp.int32),
            pltpu.VMEM((BW, VP), jnp.float32),
            pltpu.VMEM((BW, VP), jnp.float32),
            pltpu.SemaphoreType.DMA,
            pltpu.SemaphoreType.DMA,
            pltpu.SemaphoreType.DMA,
            pltpu.SemaphoreType.DMA,
        ],
    )
    def gather_k(p_hbm, idx_hbm, out_hbm, idx_v, rows_a, rows_b,
                 sem_ga, sem_gb, sem_wa, sem_wb):
        wid = lax.axis_index("s") * NC + lax.axis_index("c")
        base = wid * BW
        pltpu.sync_copy(idx_hbm.at[wid], idx_v)

        def issue_gather(l, rows, sem):
            pltpu.async_copy(p_hbm.at[idx_v.at[l]], rows, sem)

        issue_gather(0, rows_a, sem_ga)
        issue_gather(1, rows_b, sem_gb)

        def step(l, rows, sem_g, sem_w, nxt_ok):
            pltpu.make_async_copy(
                p_hbm.at[idx_v.at[l]], rows, sem_g).wait()
            pltpu.async_copy(rows, out_hbm.at[l, pl.ds(base, BW), :], sem_w)
            pltpu.make_async_copy(
                rows, out_hbm.at[l, pl.ds(base, BW), :], sem_w).wait()

            @pl.when(nxt_ok)
            def _():
                issue_gather(l + 2, rows, sem_g)

        def body(j, carry):
            ok = j + 1 < nl // 2
            step(2 * j, rows_a, sem_ga, sem_wa, ok)
            step(2 * j + 1, rows_b, sem_gb, sem_wb, ok)
            return carry

        lax.fori_loop(0, nl // 2, body, 0)

    return gather_k


def _transpose_chunk(inter, q_prev, l0, nl, bsz, seq):
    def tr(q_ref, i_ref, o_ref):
        del q_ref
        o_ref[0] = jnp.transpose(i_ref[0], (1, 0))[:V, :]

    kwargs = {}
    out_shape = jax.ShapeDtypeStruct((seq, V, bsz), jnp.float32)
    in_specs = [
        pl.BlockSpec(memory_space=pl.ANY),
        pl.BlockSpec((1, BBLK, VP), lambda l, bb: (l, bb, 0)),
    ]
    if q_prev is not None:
        kwargs["input_output_aliases"] = {0: 0}
        operands = (q_prev, inter)
    else:
        operands = (jnp.zeros((1, 1), jnp.float32), inter)
        in_specs[0] = pl.BlockSpec(memory_space=pl.ANY)
    return pl.pallas_call(
        tr,
        grid=(nl, bsz // BBLK),
        in_specs=in_specs,
        out_specs=pl.BlockSpec((1, V, BBLK), lambda l, bb: (l0 + l, 0, bb)),
        out_shape=out_shape,
        **kwargs,
    )(*operands)


def kernel(input_ids, embedding_table, linear_w, linear_b):
    bsz, seq = input_ids.shape
    w_pad = jnp.pad(linear_w, ((0, VP - V), (0, 0)))
    b_pad = jnp.pad(linear_b, (0, VP - V)).reshape(1, VP)
    p = _fused_table(embedding_table, w_pad, b_pad)
    ids3 = (input_ids.astype(jnp.int32)
            .reshape(NW, BW, seq).transpose(0, 2, 1))
    gather = _make_gather(bsz, LCH)
    inters = [gather(p, ids3[:, c * LCH:(c + 1) * LCH, :])
              for c in range(seq // LCH)]
    q = None
    for c, inter in enumerate(inters):
        q = _transpose_chunk(inter, q, c * LCH, LCH, bsz, seq)
    return lax.transpose(q, (2, 0, 1))


# speedup vs baseline: 1.6748x; 1.0775x over previous
import functools

import jax
import jax.numpy as jnp
from jax import lax
from jax.experimental import pallas as pl
from jax.experimental.pallas import tpu as pltpu
from jax.experimental.pallas import tpu_sc as plsc

V = 1000
E = 64
VP = 1024
NC = 2
NS = 16
NW = NC * NS
BW = 32
LCH = 5
BBLK = 512


def _fused_table(emb, w_pad, b_pad):
    def mm(e_ref, w_ref, b_ref, o_ref):
        o_ref[...] = lax.dot_general(
            e_ref[...], w_ref[...],
            dimension_numbers=(((1,), (1,)), ((), ())),
            preferred_element_type=jnp.float32,
        ) + b_ref[...]

    return pl.pallas_call(
        mm,
        out_shape=jax.ShapeDtypeStruct((V, VP), jnp.float32),
    )(emb, w_pad, b_pad)


def _make_gather(bsz, nl):
    mesh = plsc.VectorSubcoreMesh(core_axis_name="c", subcore_axis_name="s")

    @functools.partial(
        pl.kernel, mesh=mesh,
        out_type=jax.ShapeDtypeStruct((nl, bsz, VP), jnp.float32),
        scratch_types=[
            pltpu.VMEM((nl, BW), jnp.int32),
            pltpu.VMEM((BW, VP), jnp.float32),
            pltpu.VMEM((BW, VP), jnp.float32),
            pltpu.SemaphoreType.DMA,
            pltpu.SemaphoreType.DMA,
            pltpu.SemaphoreType.DMA,
            pltpu.SemaphoreType.DMA,
        ],
    )
    def gather_k(p_hbm, idx_hbm, out_hbm, idx_v, rows_a, rows_b,
                 sem_ga, sem_gb, sem_wa, sem_wb):
        wid = lax.axis_index("s") * NC + lax.axis_index("c")
        base = wid * BW
        pltpu.sync_copy(idx_hbm.at[wid], idx_v)

        def issue_gather(l, rows, sem):
            pltpu.async_copy(p_hbm.at[idx_v.at[l]], rows, sem)

        issue_gather(0, rows_a, sem_ga)
        issue_gather(1, rows_b, sem_gb)

        def step(l, rows, sem_g, sem_w, nxt_ok):
            pltpu.make_async_copy(
                p_hbm.at[idx_v.at[l]], rows, sem_g).wait()
            pltpu.async_copy(rows, out_hbm.at[l, pl.ds(base, BW), :], sem_w)
            pltpu.make_async_copy(
                rows, out_hbm.at[l, pl.ds(base, BW), :], sem_w).wait()

            @pl.when(nxt_ok)
            def _():
                issue_gather(l + 2, rows, sem_g)

        def body(j, carry):
            ok = j + 1 < nl // 2
            step(2 * j, rows_a, sem_ga, sem_wa, ok)
            step(2 * j + 1, rows_b, sem_gb, sem_wb, ok)
            return carry

        lax.fori_loop(0, nl // 2, body, 0)

    return gather_k


def _transpose_chunk(inter, q_prev, l0, nl, bsz, seq):
    def tr(q_ref, i_ref, o_ref):
        del q_ref
        o_ref[0] = jnp.transpose(i_ref[0], (1, 0))[:V, :]

    kwargs = {}
    out_shape = jax.ShapeDtypeStruct((seq, V, bsz), jnp.float32)
    in_specs = [
        pl.BlockSpec(memory_space=pl.ANY),
        pl.BlockSpec((1, BBLK, VP), lambda l, bb: (l, bb, 0)),
    ]
    if q_prev is not None:
        kwargs["input_output_aliases"] = {0: 0}
        operands = (q_prev, inter)
    else:
        operands = (jnp.zeros((1, 1), jnp.float32), inter)
        in_specs[0] = pl.BlockSpec(memory_space=pl.ANY)
    return pl.pallas_call(
        tr,
        grid=(nl, bsz // BBLK),
        in_specs=in_specs,
        out_specs=pl.BlockSpec((1, V, BBLK), lambda l, bb: (l0 + l, 0, bb)),
        out_shape=out_shape,
        **kwargs,
    )(*operands)


def kernel(input_ids, embedding_table, linear_w, linear_b):
    bsz, seq = input_ids.shape
    w_pad = jnp.pad(linear_w, ((0, VP - V), (0, 0)))
    b_pad = jnp.pad(linear_b, (0, VP - V)).reshape(1, VP)
    p = _fused_table(embedding_table, w_pad, b_pad)
    ids3 = (input_ids.astype(jnp.int32)
            .reshape(NW, BW, seq).transpose(0, 2, 1))
    gather = _make_gather(bsz, LCH)
    inters = [gather(p, ids3[:, c * LCH:(c + 1) * LCH, :])
              for c in range(seq // LCH)]
    q = None
    for c, inter in enumerate(inters):
        q = _transpose_chunk(inter, q, c * LCH, LCH, bsz, seq)
    return lax.transpose(q, (2, 0, 1))
